# baseline (speedup 1.0000x reference)
import functools

import jax
import jax.numpy as jnp
from jax import lax
from jax.experimental import pallas as pl
from jax.experimental.pallas import tpu as pltpu
from jax.experimental.pallas import tpu_sc as plsc

NUM_TABLES = 26
VOCAB = 100000
EMBED_DIM = 16
BATCH = 16384
DENSE_OUT = 5

_info = plsc.get_sparse_core_info()
_NC, _NS = _info.num_cores, _info.num_subcores
NW = _NC * _NS
BPW = BATCH // NW

_MAIN = 768
_TAILR = VOCAB - _MAIN * 128

_MESH = plsc.VectorSubcoreMesh(core_axis_name="c", subcore_axis_name="s")


_SCN = 11
_G = 12
_GC = _G * 128
_GW = _GC * EMBED_DIM
_BPWK = _MAIN // NW
_NSTEP = _SCN * (_BPWK // _G)


def _transpose_group(blk_v, rows_v, ncols):
    lanes = lax.iota(jnp.int32, 16)

    @plsc.parallel_loop(0, ncols, unroll=16)
    def col(c):
        row = plsc.load_gather(blk_v, [lanes, jnp.full((16,), c, jnp.int32)])
        rows_v[pl.ds(c * EMBED_DIM, EMBED_DIM)] = row


def _sc_convert(tails, *refs):
    t_phys = refs[:_SCN]
    outs = refs[_SCN:2 * _SCN]
    blk = refs[2 * _SCN:2 * _SCN + 2]
    rows = refs[2 * _SCN + 2:2 * _SCN + 4]
    in_sem = refs[2 * _SCN + 4:2 * _SCN + 6]
    out_sem = refs[2 * _SCN + 6:2 * _SCN + 8]
    wid = lax.axis_index("s") * _NC + lax.axis_index("c")
    col0 = wid * (_BPWK * 128)

    def step_src(s):
        t, g = divmod(s, _BPWK // _G)
        return t_phys[t].at[:, pl.ds(col0 + g * _GC, _GC)]

    def step_dst(s):
        t, g = divmod(s, _BPWK // _G)
        return outs[t].at[pl.ds(col0 * EMBED_DIM + g * _GW, _GW)]

    in_copies = [None, None]
    out_copies = [None, None]
    for s in range(_NSTEP + 1):
        p = s % 2
        if s < _NSTEP:
            in_copies[p] = pltpu.async_copy(
                step_src(s), blk[p].at[:, pl.ds(0, _GC)], in_sem[p])
        if s > 0:
            q = (s - 1) % 2
            in_copies[q].wait()
            if out_copies[q] is not None:
                out_copies[q].wait()
            _transpose_group(blk[q], rows[q], _GC)
            out_copies[q] = pltpu.async_copy(rows[q], step_dst(s - 1),
                                             out_sem[q])
    for c in out_copies:
        if c is not None:
            c.wait()

    for t in range(_SCN):
        @pl.when(wid == t)
        def _(t=t):
            pltpu.sync_copy(tails.at[pl.ds(t * _TAILR * EMBED_DIM,
                                           _TAILR * EMBED_DIM)],
                            outs[t].at[pl.ds(_MAIN * 2048,
                                             _TAILR * EMBED_DIM)])


_convert_call = functools.partial(
    pl.kernel,
    mesh=_MESH,
    compiler_params=pltpu.CompilerParams(use_tc_tiling_on_sc=True,
                                         needs_layout_passes=False),
    out_type=[jax.ShapeDtypeStruct((VOCAB * EMBED_DIM,), jnp.float32)
              for _ in range(_SCN)],
    scratch_types=[
        pltpu.VMEM((EMBED_DIM, _GC + 1), jnp.float32),
        pltpu.VMEM((EMBED_DIM, _GC + 1), jnp.float32),
        pltpu.VMEM((_GW,), jnp.float32),
        pltpu.VMEM((_GW,), jnp.float32),
        pltpu.SemaphoreType.DMA,
        pltpu.SemaphoreType.DMA,
        pltpu.SemaphoreType.DMA,
        pltpu.SemaphoreType.DMA,
    ],
)(_sc_convert)


def _sc_gather(idx_hbm, *rest):
    tables = rest[:NUM_TABLES]
    out_hbm = rest[NUM_TABLES]
    idx_v, rows_v, sem = rest[NUM_TABLES + 1:]
    wid = lax.axis_index("s") * _NC + lax.axis_index("c")
    base = wid * BPW
    for t in range(NUM_TABLES):
        pltpu.sync_copy(idx_hbm.at[t, pl.ds(base, BPW)], idx_v)
        pltpu.async_copy(tables[t].at[idx_v], rows_v, sem).wait()
        pltpu.sync_copy(rows_v, out_hbm.at[t, pl.ds(base, BPW)])


_gather_call = functools.partial(
    pl.kernel,
    mesh=_MESH,
    compiler_params=pltpu.CompilerParams(use_tc_tiling_on_sc=False),
    out_type=jax.ShapeDtypeStruct((NUM_TABLES, BATCH, EMBED_DIM), jnp.float32),
    scratch_types=[
        pltpu.VMEM((BPW,), jnp.int32),
        pltpu.VMEM((BPW, EMBED_DIM), jnp.float32),
        pltpu.SemaphoreType.DMA,
    ],
)(_sc_gather)


_BT = 256
_PACK = 128 // EMBED_DIM
_NOUT = _PACK * DENSE_OUT


def _mm_body(x_ref, w_ref, b_ref, o_ref):
    acc = jnp.zeros((_BT, _NOUT), jnp.float32)
    for t in range(NUM_TABLES):
        acc = acc + lax.dot_general(
            x_ref[t], w_ref[t], (((1,), (0,)), ((), ())),
            preferred_element_type=jnp.float32)
    o_ref[...] = acc + b_ref[...]


def _dense(x3p, wp, b2):
    nrows = BATCH // _PACK
    return pl.pallas_call(
        _mm_body,
        grid=(nrows // _BT,),
        in_specs=[
            pl.BlockSpec((NUM_TABLES, _BT, 128), lambda i: (0, i, 0)),
            pl.BlockSpec((NUM_TABLES, 128, _NOUT), lambda i: (0, 0, 0)),
            pl.BlockSpec((1, _NOUT), lambda i: (0, 0)),
        ],
        out_specs=pl.BlockSpec((_BT, _NOUT), lambda i: (i, 0)),
        out_shape=jax.ShapeDtypeStruct((nrows, _NOUT), jnp.float32),
    )(x3p, wp, b2)


def kernel(idx_0, idx_1, idx_2, idx_3, idx_4, idx_5, idx_6, idx_7, idx_8, idx_9, idx_10, idx_11, idx_12, idx_13, idx_14, idx_15, idx_16, idx_17, idx_18, idx_19, idx_20, idx_21, idx_22, idx_23, idx_24, idx_25, table_0, table_1, table_2, table_3, table_4, table_5, table_6, table_7, table_8, table_9, table_10, table_11, table_12, table_13, table_14, table_15, table_16, table_17, table_18, table_19, table_20, table_21, table_22, table_23, table_24, table_25, dense_w, dense_b):
    idxs = [idx_0, idx_1, idx_2, idx_3, idx_4, idx_5, idx_6, idx_7, idx_8, idx_9,
            idx_10, idx_11, idx_12, idx_13, idx_14, idx_15, idx_16, idx_17, idx_18,
            idx_19, idx_20, idx_21, idx_22, idx_23, idx_24, idx_25]
    tables = [table_0, table_1, table_2, table_3, table_4, table_5, table_6,
              table_7, table_8, table_9, table_10, table_11, table_12, table_13,
              table_14, table_15, table_16, table_17, table_18, table_19,
              table_20, table_21, table_22, table_23, table_24, table_25]
    idx_stack = jnp.stack(idxs, axis=0)
    t_phys = [jnp.swapaxes(t, 0, 1) for t in tables]
    tails = jnp.stack([t[_MAIN * 128:] for t in tables[:_SCN]],
                      axis=0).reshape(-1)
    tables_rm = _convert_call(tails, *t_phys[:_SCN])
    tables_2d = ([t.reshape(VOCAB, EMBED_DIM) for t in tables_rm]
                 + list(tables[_SCN:]))
    x3 = _gather_call(idx_stack, *tables_2d)
    x3p = x3.reshape(NUM_TABLES, BATCH // _PACK, 128)
    w3 = dense_w.reshape(NUM_TABLES, EMBED_DIM, DENSE_OUT)
    eye8 = jnp.eye(_PACK, dtype=jnp.float32)
    wp = jax.vmap(lambda m: jnp.kron(eye8, m))(w3)
    b2 = jnp.tile(dense_b, _PACK).reshape(1, _NOUT)
    out_packed = _dense(x3p, wp, b2)
    return out_packed.reshape(BATCH, DENSE_OUT)


# speedup vs baseline: 1.0729x; 1.0729x over previous
import functools

import jax
import jax.numpy as jnp
from jax import lax
from jax.experimental import pallas as pl
from jax.experimental.pallas import tpu as pltpu
from jax.experimental.pallas import tpu_sc as plsc

NUM_TABLES = 26
VOCAB = 100000
EMBED_DIM = 16
BATCH = 16384
DENSE_OUT = 5

_info = plsc.get_sparse_core_info()
_NC, _NS = _info.num_cores, _info.num_subcores
NW = _NC * _NS
BPW = BATCH // NW

_MAIN = 768
_TAILR = VOCAB - _MAIN * 128

_MESH = plsc.VectorSubcoreMesh(core_axis_name="c", subcore_axis_name="s")


_SCN = 14
_G = 12
_GC = _G * 128
_GW = _GC * EMBED_DIM
_BPWK = _MAIN // NW
_NSTEP = _SCN * (_BPWK // _G)


def _transpose_group(blk_v, rows_v, ncols):
    lanes = lax.iota(jnp.int32, 16)

    @plsc.parallel_loop(0, ncols, unroll=16)
    def col(c):
        row = plsc.load_gather(blk_v, [lanes, jnp.full((16,), c, jnp.int32)])
        rows_v[pl.ds(c * EMBED_DIM, EMBED_DIM)] = row


def _sc_convert(tails, *refs):
    t_phys = refs[:_SCN]
    outs = refs[_SCN:2 * _SCN]
    blk = refs[2 * _SCN:2 * _SCN + 2]
    rows = refs[2 * _SCN + 2:2 * _SCN + 4]
    in_sem = refs[2 * _SCN + 4:2 * _SCN + 6]
    out_sem = refs[2 * _SCN + 6:2 * _SCN + 8]
    wid = lax.axis_index("s") * _NC + lax.axis_index("c")
    col0 = wid * (_BPWK * 128)

    def step_src(s):
        t, g = divmod(s, _BPWK // _G)
        return t_phys[t].at[:, pl.ds(col0 + g * _GC, _GC)]

    def step_dst(s):
        t, g = divmod(s, _BPWK // _G)
        return outs[t].at[pl.ds(col0 * EMBED_DIM + g * _GW, _GW)]

    in_copies = [None, None]
    out_copies = [None, None]
    for s in range(_NSTEP + 1):
        p = s % 2
        if s < _NSTEP:
            in_copies[p] = pltpu.async_copy(
                step_src(s), blk[p].at[:, pl.ds(0, _GC)], in_sem[p])
        if s > 0:
            q = (s - 1) % 2
            in_copies[q].wait()
            if out_copies[q] is not None:
                out_copies[q].wait()
            _transpose_group(blk[q], rows[q], _GC)
            out_copies[q] = pltpu.async_copy(rows[q], step_dst(s - 1),
                                             out_sem[q])
    for c in out_copies:
        if c is not None:
            c.wait()

    for t in range(_SCN):
        @pl.when(wid == t)
        def _(t=t):
            pltpu.sync_copy(tails.at[pl.ds(t * _TAILR * EMBED_DIM,
                                           _TAILR * EMBED_DIM)],
                            outs[t].at[pl.ds(_MAIN * 2048,
                                             _TAILR * EMBED_DIM)])


_convert_call = functools.partial(
    pl.kernel,
    mesh=_MESH,
    compiler_params=pltpu.CompilerParams(use_tc_tiling_on_sc=True,
                                         needs_layout_passes=False),
    out_type=[jax.ShapeDtypeStruct((VOCAB * EMBED_DIM,), jnp.float32)
              for _ in range(_SCN)],
    scratch_types=[
        pltpu.VMEM((EMBED_DIM, _GC + 1), jnp.float32),
        pltpu.VMEM((EMBED_DIM, _GC + 1), jnp.float32),
        pltpu.VMEM((_GW,), jnp.float32),
        pltpu.VMEM((_GW,), jnp.float32),
        pltpu.SemaphoreType.DMA,
        pltpu.SemaphoreType.DMA,
        pltpu.SemaphoreType.DMA,
        pltpu.SemaphoreType.DMA,
    ],
)(_sc_convert)


def _sc_gather(idx_hbm, *rest):
    tables = rest[:NUM_TABLES]
    out_hbm = rest[NUM_TABLES]
    idx_v, rows_v, sem = rest[NUM_TABLES + 1:]
    wid = lax.axis_index("s") * _NC + lax.axis_index("c")
    base = wid * BPW
    for t in range(NUM_TABLES):
        pltpu.sync_copy(idx_hbm.at[t, pl.ds(base, BPW)], idx_v)
        pltpu.async_copy(tables[t].at[idx_v], rows_v, sem).wait()
        pltpu.sync_copy(rows_v, out_hbm.at[t, pl.ds(base, BPW)])


_gather_call = functools.partial(
    pl.kernel,
    mesh=_MESH,
    compiler_params=pltpu.CompilerParams(use_tc_tiling_on_sc=False),
    out_type=jax.ShapeDtypeStruct((NUM_TABLES, BATCH, EMBED_DIM), jnp.float32),
    scratch_types=[
        pltpu.VMEM((BPW,), jnp.int32),
        pltpu.VMEM((BPW, EMBED_DIM), jnp.float32),
        pltpu.SemaphoreType.DMA,
    ],
)(_sc_gather)


_BT = 256
_PACK = 128 // EMBED_DIM
_NOUT = _PACK * DENSE_OUT


def _mm_body(x_ref, w_ref, b_ref, o_ref):
    acc = jnp.zeros((_BT, _NOUT), jnp.float32)
    for t in range(NUM_TABLES):
        acc = acc + lax.dot_general(
            x_ref[t], w_ref[t], (((1,), (0,)), ((), ())),
            preferred_element_type=jnp.float32)
    o_ref[...] = acc + b_ref[...]


def _dense(x3p, wp, b2):
    nrows = BATCH // _PACK
    return pl.pallas_call(
        _mm_body,
        grid=(nrows // _BT,),
        in_specs=[
            pl.BlockSpec((NUM_TABLES, _BT, 128), lambda i: (0, i, 0)),
            pl.BlockSpec((NUM_TABLES, 128, _NOUT), lambda i: (0, 0, 0)),
            pl.BlockSpec((1, _NOUT), lambda i: (0, 0)),
        ],
        out_specs=pl.BlockSpec((_BT, _NOUT), lambda i: (i, 0)),
        out_shape=jax.ShapeDtypeStruct((nrows, _NOUT), jnp.float32),
    )(x3p, wp, b2)


def kernel(idx_0, idx_1, idx_2, idx_3, idx_4, idx_5, idx_6, idx_7, idx_8, idx_9, idx_10, idx_11, idx_12, idx_13, idx_14, idx_15, idx_16, idx_17, idx_18, idx_19, idx_20, idx_21, idx_22, idx_23, idx_24, idx_25, table_0, table_1, table_2, table_3, table_4, table_5, table_6, table_7, table_8, table_9, table_10, table_11, table_12, table_13, table_14, table_15, table_16, table_17, table_18, table_19, table_20, table_21, table_22, table_23, table_24, table_25, dense_w, dense_b):
    idxs = [idx_0, idx_1, idx_2, idx_3, idx_4, idx_5, idx_6, idx_7, idx_8, idx_9,
            idx_10, idx_11, idx_12, idx_13, idx_14, idx_15, idx_16, idx_17, idx_18,
            idx_19, idx_20, idx_21, idx_22, idx_23, idx_24, idx_25]
    tables = [table_0, table_1, table_2, table_3, table_4, table_5, table_6,
              table_7, table_8, table_9, table_10, table_11, table_12, table_13,
              table_14, table_15, table_16, table_17, table_18, table_19,
              table_20, table_21, table_22, table_23, table_24, table_25]
    idx_stack = jnp.stack(idxs, axis=0)
    t_phys = [jnp.swapaxes(t, 0, 1) for t in tables]
    tails = jnp.stack([t[_MAIN * 128:] for t in tables[:_SCN]],
                      axis=0).reshape(-1)
    tables_rm = _convert_call(tails, *t_phys[:_SCN])
    tables_2d = ([t.reshape(VOCAB, EMBED_DIM) for t in tables_rm]
                 + list(tables[_SCN:]))
    x3 = _gather_call(idx_stack, *tables_2d)
    x3p = x3.reshape(NUM_TABLES, BATCH // _PACK, 128)
    w3 = dense_w.reshape(NUM_TABLES, EMBED_DIM, DENSE_OUT)
    eye8 = jnp.eye(_PACK, dtype=jnp.float32)
    wp = jax.vmap(lambda m: jnp.kron(eye8, m))(w3)
    b2 = jnp.tile(dense_b, _PACK).reshape(1, _NOUT)
    out_packed = _dense(x3p, wp, b2)
    return out_packed.reshape(BATCH, DENSE_OUT)


# speedup vs baseline: 1.1460x; 1.0681x over previous
import functools

import jax
import jax.numpy as jnp
from jax import lax
from jax.experimental import pallas as pl
from jax.experimental.pallas import tpu as pltpu
from jax.experimental.pallas import tpu_sc as plsc

NUM_TABLES = 26
VOCAB = 100000
EMBED_DIM = 16
BATCH = 16384
DENSE_OUT = 5

_info = plsc.get_sparse_core_info()
_NC, _NS = _info.num_cores, _info.num_subcores
NW = _NC * _NS
BPW = BATCH // NW

_MAIN = 768
_TAILR = VOCAB - _MAIN * 128

_MESH = plsc.VectorSubcoreMesh(core_axis_name="c", subcore_axis_name="s")


_SCN = 13
_G = 12
_GC = _G * 128
_GW = _GC * EMBED_DIM
_BPWK = _MAIN // NW
_NSTEP = _SCN * (_BPWK // _G)


def _transpose_group(blk_v, rows_v, ncols):
    lanes = lax.iota(jnp.int32, 16)

    @plsc.parallel_loop(0, ncols, unroll=16)
    def col(c):
        row = plsc.load_gather(blk_v, [lanes, jnp.full((16,), c, jnp.int32)])
        rows_v[pl.ds(c * EMBED_DIM, EMBED_DIM)] = row


def _sc_convert(tails, *refs):
    t_phys = refs[:_SCN]
    outs = refs[_SCN:2 * _SCN]
    blk = refs[2 * _SCN:2 * _SCN + 2]
    rows = refs[2 * _SCN + 2:2 * _SCN + 4]
    in_sem = refs[2 * _SCN + 4:2 * _SCN + 6]
    out_sem = refs[2 * _SCN + 6:2 * _SCN + 8]
    wid = lax.axis_index("s") * _NC + lax.axis_index("c")
    col0 = wid * (_BPWK * 128)

    def step_src(s):
        t, g = divmod(s, _BPWK // _G)
        return t_phys[t].at[:, pl.ds(col0 + g * _GC, _GC)]

    def step_dst(s):
        t, g = divmod(s, _BPWK // _G)
        return outs[t].at[pl.ds(col0 * EMBED_DIM + g * _GW, _GW)]

    in_copies = [None, None]
    out_copies = [None, None]
    for s in range(_NSTEP + 1):
        p = s % 2
        if s < _NSTEP:
            in_copies[p] = pltpu.async_copy(
                step_src(s), blk[p].at[:, pl.ds(0, _GC)], in_sem[p])
        if s > 0:
            q = (s - 1) % 2
            in_copies[q].wait()
            if out_copies[q] is not None:
                out_copies[q].wait()
            _transpose_group(blk[q], rows[q], _GC)
            out_copies[q] = pltpu.async_copy(rows[q], step_dst(s - 1),
                                             out_sem[q])
    for c in out_copies:
        if c is not None:
            c.wait()

    for t in range(_SCN):
        @pl.when(wid == t)
        def _(t=t):
            pltpu.sync_copy(tails.at[pl.ds(t * _TAILR * EMBED_DIM,
                                           _TAILR * EMBED_DIM)],
                            outs[t].at[pl.ds(_MAIN * 2048,
                                             _TAILR * EMBED_DIM)])


_convert_call = functools.partial(
    pl.kernel,
    mesh=_MESH,
    compiler_params=pltpu.CompilerParams(use_tc_tiling_on_sc=True,
                                         needs_layout_passes=False),
    out_type=[jax.ShapeDtypeStruct((VOCAB * EMBED_DIM,), jnp.float32)
              for _ in range(_SCN)],
    scratch_types=[
        pltpu.VMEM((EMBED_DIM, _GC + 1), jnp.float32),
        pltpu.VMEM((EMBED_DIM, _GC + 1), jnp.float32),
        pltpu.VMEM((_GW,), jnp.float32),
        pltpu.VMEM((_GW,), jnp.float32),
        pltpu.SemaphoreType.DMA,
        pltpu.SemaphoreType.DMA,
        pltpu.SemaphoreType.DMA,
        pltpu.SemaphoreType.DMA,
    ],
)(_sc_convert)


def _sc_gather(idx_hbm, *rest):
    tables = rest[:NUM_TABLES]
    out_hbm = rest[NUM_TABLES]
    idx_v = rest[NUM_TABLES + 1]
    rows = rest[NUM_TABLES + 2:NUM_TABLES + 4]
    g_sem = rest[NUM_TABLES + 4:NUM_TABLES + 6]
    w_sem = rest[NUM_TABLES + 6:NUM_TABLES + 8]
    wid = lax.axis_index("s") * _NC + lax.axis_index("c")
    base = wid * BPW
    pltpu.sync_copy(idx_hbm.at[:, pl.ds(base, BPW)], idx_v)
    gathers = [None, None]
    writes = [None, None]
    for t in range(NUM_TABLES + 1):
        p = t % 2
        if t < NUM_TABLES:
            if writes[p] is not None:
                writes[p].wait()
            gathers[p] = pltpu.async_copy(tables[t].at[idx_v.at[t]], rows[p],
                                          g_sem[p])
        if t > 0:
            q = (t - 1) % 2
            gathers[q].wait()
            writes[q] = pltpu.async_copy(
                rows[q], out_hbm.at[t - 1, pl.ds(base, BPW)], w_sem[q])
    for w in writes:
        if w is not None:
            w.wait()


_gather_call = functools.partial(
    pl.kernel,
    mesh=_MESH,
    compiler_params=pltpu.CompilerParams(use_tc_tiling_on_sc=False),
    out_type=jax.ShapeDtypeStruct((NUM_TABLES, BATCH, EMBED_DIM), jnp.float32),
    scratch_types=[
        pltpu.VMEM((NUM_TABLES, BPW), jnp.int32),
        pltpu.VMEM((BPW, EMBED_DIM), jnp.float32),
        pltpu.VMEM((BPW, EMBED_DIM), jnp.float32),
        pltpu.SemaphoreType.DMA,
        pltpu.SemaphoreType.DMA,
        pltpu.SemaphoreType.DMA,
        pltpu.SemaphoreType.DMA,
    ],
)(_sc_gather)


_BT = 256
_PACK = 128 // EMBED_DIM
_NOUT = _PACK * DENSE_OUT


def _mm_body(x_ref, w_ref, b_ref, o_ref):
    acc = jnp.zeros((_BT, _NOUT), jnp.float32)
    for t in range(NUM_TABLES):
        acc = acc + lax.dot_general(
            x_ref[t], w_ref[t], (((1,), (0,)), ((), ())),
            preferred_element_type=jnp.float32)
    o_ref[...] = acc + b_ref[...]


def _dense(x3p, wp, b2):
    nrows = BATCH // _PACK
    return pl.pallas_call(
        _mm_body,
        grid=(nrows // _BT,),
        in_specs=[
            pl.BlockSpec((NUM_TABLES, _BT, 128), lambda i: (0, i, 0)),
            pl.BlockSpec((NUM_TABLES, 128, _NOUT), lambda i: (0, 0, 0)),
            pl.BlockSpec((1, _NOUT), lambda i: (0, 0)),
        ],
        out_specs=pl.BlockSpec((_BT, _NOUT), lambda i: (i, 0)),
        out_shape=jax.ShapeDtypeStruct((nrows, _NOUT), jnp.float32),
    )(x3p, wp, b2)


def kernel(idx_0, idx_1, idx_2, idx_3, idx_4, idx_5, idx_6, idx_7, idx_8, idx_9, idx_10, idx_11, idx_12, idx_13, idx_14, idx_15, idx_16, idx_17, idx_18, idx_19, idx_20, idx_21, idx_22, idx_23, idx_24, idx_25, table_0, table_1, table_2, table_3, table_4, table_5, table_6, table_7, table_8, table_9, table_10, table_11, table_12, table_13, table_14, table_15, table_16, table_17, table_18, table_19, table_20, table_21, table_22, table_23, table_24, table_25, dense_w, dense_b):
    idxs = [idx_0, idx_1, idx_2, idx_3, idx_4, idx_5, idx_6, idx_7, idx_8, idx_9,
            idx_10, idx_11, idx_12, idx_13, idx_14, idx_15, idx_16, idx_17, idx_18,
            idx_19, idx_20, idx_21, idx_22, idx_23, idx_24, idx_25]
    tables = [table_0, table_1, table_2, table_3, table_4, table_5, table_6,
              table_7, table_8, table_9, table_10, table_11, table_12, table_13,
              table_14, table_15, table_16, table_17, table_18, table_19,
              table_20, table_21, table_22, table_23, table_24, table_25]
    idx_stack = jnp.stack(idxs, axis=0)
    t_phys = [jnp.swapaxes(t, 0, 1) for t in tables]
    tails = jnp.stack([t[_MAIN * 128:] for t in tables[:_SCN]],
                      axis=0).reshape(-1)
    tables_rm = _convert_call(tails, *t_phys[:_SCN])
    tables_2d = ([t.reshape(VOCAB, EMBED_DIM) for t in tables_rm]
                 + list(tables[_SCN:]))
    x3 = _gather_call(idx_stack, *tables_2d)
    x3p = x3.reshape(NUM_TABLES, BATCH // _PACK, 128)
    w3 = dense_w.reshape(NUM_TABLES, EMBED_DIM, DENSE_OUT)
    eye8 = jnp.eye(_PACK, dtype=jnp.float32)
    wp = jax.vmap(lambda m: jnp.kron(eye8, m))(w3)
    b2 = jnp.tile(dense_b, _PACK).reshape(1, _NOUT)
    out_packed = _dense(x3p, wp, b2)
    return out_packed.reshape(BATCH, DENSE_OUT)
